# Initial kernel scaffold; baseline (speedup 1.0000x reference)
import jax
import jax.numpy as jnp
from jax.experimental import pallas as pl


def kernel(pred_similarities, regions_mask, kernels_mask, text_mask_ndi_labels, kernel_mask_ndi_labels):
    raise NotImplementedError("write your pallas kernel here")


# speedup vs baseline: 143.9344x; 143.9344x over previous
import jax
import jax.numpy as jnp
from jax.experimental import pallas as pl
from jax.experimental.pallas import tpu as pltpu

_NL = 8
_SIG = 0.5


def _sums_body(pred_ref, km_ref, kl_ref, rl_ref, out_ref):
    h = pl.program_id(1)
    kl = kl_ref[0, 0]
    rl = rl_ref[0, 0]
    km = km_ref[0, 0]
    i0 = jax.lax.broadcasted_iota(jnp.int32, (_NL, _NL), 0)
    i1 = jax.lax.broadcasted_iota(jnp.int32, (_NL, _NL), 1)
    tbl = jnp.zeros((_NL, _NL), jnp.float32)
    for l in range(_NL):
        mk = (kl == l).astype(jnp.float32)
        sk = jnp.sum(mk * km)
        tbl = jnp.where((i0 == 0) & (i1 == l), sk, tbl)
        for c in range(4):
            sc = jnp.sum(mk * pred_ref[0, c])
            tbl = jnp.where((i0 == 1 + c) & (i1 == l), sc, tbl)
        sr = jnp.sum(jnp.where(rl == l, km, 0.0))
        tbl = jnp.where((i0 == 5) & (i1 == l), sr, tbl)
    mx = jnp.max(kl).astype(jnp.float32)
    tbl = jnp.where(i0 == 6, mx, tbl)

    @pl.when(h == 0)
    def _():
        out_ref[0] = tbl

    @pl.when(h != 0)
    def _():
        prev = out_ref[0]
        out_ref[0] = jnp.where(i0 == 6, jnp.maximum(prev, tbl), prev + tbl)


def _loss_body(tbl_ref, pred_ref, rm_ref, kl_ref, rl_ref, out_ref):
    b = pl.program_id(0)
    h = pl.program_id(1)
    kl = kl_ref[0, 0]
    rl = rl_ref[0, 0]
    rm = rm_ref[0, 0]

    acc = jnp.zeros_like(rm)
    for c in range(4):
        fp = pred_ref[0, c] * rm
        gk = jnp.zeros_like(fp)
        for l in range(1, _NL):
            ks = tbl_ref[0, 0, l]
            cs = tbl_ref[0, 1 + c, l]
            g = cs / (ks + 1.0)
            gk = jnp.where(kl == l, g, gk)
        d = fp - gk
        acc = acc + d * d
    norm = jnp.sqrt(acc) - _SIG
    dd = jnp.maximum(norm, 0.0)
    dd = jnp.log(dd * dd + 1.0)
    rg = jnp.ones_like(rm)
    for l in range(1, _NL):
        rs = tbl_ref[0, 5, l]
        rg = jnp.where(rl == l, 1.0 / (rs + 1.0), rg)
    s = jnp.sum(dd * rg)

    first = jnp.logical_and(b == 0, h == 0)

    @pl.when(first)
    def _():
        out_ref[0, 0] = s

    @pl.when(jnp.logical_not(first))
    def _():
        out_ref[0, 0] = out_ref[0, 0] + s


def kernel(pred_similarities, regions_mask, kernels_mask, text_mask_ndi_labels, kernel_mask_ndi_labels):
    B, C, H, W = pred_similarities.shape
    Rb = 128
    nh = H // Rb

    img_spec = lambda: pl.BlockSpec((1, 1, Rb, W), lambda b, h: (b, 0, h, 0))
    pred_spec = pl.BlockSpec((1, C, Rb, W), lambda b, h: (b, 0, h, 0))

    tbl = pl.pallas_call(
        _sums_body,
        grid=(B, nh),
        in_specs=[pred_spec, img_spec(), img_spec(), img_spec()],
        out_specs=pl.BlockSpec((1, _NL, _NL), lambda b, h: (b, 0, 0)),
        out_shape=jax.ShapeDtypeStruct((B, _NL, _NL), jnp.float32),
    )(pred_similarities, kernels_mask, kernel_mask_ndi_labels, text_mask_ndi_labels)

    loss_sum = pl.pallas_call(
        _loss_body,
        grid=(B, nh),
        in_specs=[
            pl.BlockSpec((1, _NL, _NL), lambda b, h: (b, 0, 0)),
            pred_spec,
            img_spec(),
            img_spec(),
            img_spec(),
        ],
        out_specs=pl.BlockSpec(memory_space=pltpu.SMEM),
        out_shape=jax.ShapeDtypeStruct((1, 1), jnp.float32),
    )(tbl, pred_similarities, regions_mask, kernel_mask_ndi_labels, text_mask_ndi_labels)

    num_kernel = tbl[B - 1, 6, 0]
    return loss_sum[0, 0] / num_kernel


# speedup vs baseline: 175.5283x; 1.2195x over previous
import jax
import jax.numpy as jnp
from jax.experimental import pallas as pl
from jax.experimental.pallas import tpu as pltpu

_NL = 8
_SIG = 0.5


def _body(pred_ref, rm_ref, km_ref, rl_ref, kl_ref, loss_ref, numk_ref):
    b = pl.program_id(0)
    nb = pl.num_programs(0)
    kl = kl_ref[0, 0]
    rl = rl_ref[0, 0]
    km = km_ref[0, 0]
    rm = rm_ref[0, 0]
    preds = [pred_ref[0, c] for c in range(4)]

    kmask = [kl == l for l in range(1, _NL)]
    rmask = [rl == l for l in range(1, _NL)]
    zero = jnp.zeros_like(km)
    ks = [jnp.sum(jnp.where(m, km, zero)) for m in kmask]
    cs = [[jnp.sum(jnp.where(m, p, zero)) for m in kmask] for p in preds]
    rs = [jnp.sum(jnp.where(m, km, zero)) for m in rmask]
    inv_k = [1.0 / (s + 1.0) for s in ks]
    g = [[cs[c][i] * inv_k[i] for i in range(7)] for c in range(4)]
    rinv = [1.0 / (s + 1.0) for s in rs]

    acc = jnp.zeros_like(km)
    for c in range(4):
        fp = preds[c] * rm
        gk = zero
        for i in range(7):
            gk = jnp.where(kmask[i], g[c][i], gk)
        d = fp - gk
        acc = acc + d * d
    dd = jnp.maximum(jnp.sqrt(acc) - _SIG, 0.0)
    dd = jnp.log(dd * dd + 1.0)
    rg = jnp.ones_like(km)
    for i in range(7):
        rg = jnp.where(rmask[i], rinv[i], rg)
    s = jnp.sum(dd * rg)

    @pl.when(b == 0)
    def _():
        loss_ref[0, 0] = s

    @pl.when(b != 0)
    def _():
        loss_ref[0, 0] = loss_ref[0, 0] + s

    @pl.when(b == nb - 1)
    def _():
        numk_ref[0, 0] = jnp.max(kl).astype(jnp.float32)


def kernel(pred_similarities, regions_mask, kernels_mask, text_mask_ndi_labels, kernel_mask_ndi_labels):
    B, C, H, W = pred_similarities.shape

    img_spec = lambda: pl.BlockSpec((1, 1, H, W), lambda b: (b, 0, 0, 0))
    scal_spec = lambda: pl.BlockSpec(memory_space=pltpu.SMEM)

    loss_sum, numk = pl.pallas_call(
        _body,
        grid=(B,),
        in_specs=[
            pl.BlockSpec((1, C, H, W), lambda b: (b, 0, 0, 0)),
            img_spec(),
            img_spec(),
            img_spec(),
            img_spec(),
        ],
        out_specs=[scal_spec(), scal_spec()],
        out_shape=[
            jax.ShapeDtypeStruct((1, 1), jnp.float32),
            jax.ShapeDtypeStruct((1, 1), jnp.float32),
        ],
    )(pred_similarities, regions_mask, kernels_mask, text_mask_ndi_labels, kernel_mask_ndi_labels)

    return loss_sum[0, 0] / numk[0, 0]


# speedup vs baseline: 217.7779x; 1.2407x over previous
import jax
import jax.numpy as jnp
from jax.experimental import pallas as pl
from jax.experimental.pallas import tpu as pltpu

_NL = 8
_SIG = 0.5


def _body(pred_ref, rm_ref, km_ref, rl_ref, kl_ref, loss_ref):
    b = pl.program_id(0)
    nb = pl.num_programs(0)
    kl = kl_ref[0, 0]
    rl = rl_ref[0, 0]
    km = km_ref[0, 0]
    rm = rm_ref[0, 0]
    H = kl.shape[0]
    preds = [pred_ref[0, c] for c in range(4)]

    zero = jnp.zeros_like(km)
    z11 = jnp.zeros((1, 1), jnp.float32)

    def msum(mask, data):
        return jnp.sum(jnp.where(mask, data, zero), axis=(0, 1), keepdims=True)

    kmask = [kl == l for l in range(1, _NL)]
    rmask = [rl == l for l in range(1, _NL)]
    ks_t = jnp.concatenate([z11] + [msum(m, km) for m in kmask], axis=1)
    rs_t = jnp.concatenate([z11] + [msum(m, km) for m in rmask], axis=1)
    cs_t = [jnp.concatenate([z11] + [msum(m, p) for m in kmask], axis=1)
            for p in preds]

    inv_k = 1.0 / (ks_t + 1.0)
    g_t = [c * inv_k for c in cs_t]
    lane = jax.lax.broadcasted_iota(jnp.int32, (1, _NL), 1)
    rinv_t = jnp.where(lane > 0, 1.0 / (rs_t + 1.0), 1.0)

    def gather(t, idx):
        tb = jnp.broadcast_to(t, (H, _NL))
        return jnp.take_along_axis(tb, idx, axis=1, mode="promise_in_bounds")

    acc = zero
    for c in range(4):
        fp = preds[c] * rm
        d = fp - gather(g_t[c], kl)
        acc = acc + d * d
    dd = jnp.maximum(jnp.sqrt(acc) - _SIG, 0.0)
    dd = jnp.log(dd * dd + 1.0)
    s = jnp.sum(dd * gather(rinv_t, rl))

    @pl.when(b == 0)
    def _():
        loss_ref[0, 0] = s

    @pl.when(jnp.logical_and(b != 0, b != nb - 1))
    def _():
        loss_ref[0, 0] = loss_ref[0, 0] + s

    @pl.when(jnp.logical_and(b != 0, b == nb - 1))
    def _():
        numk = jnp.max(kl).astype(jnp.float32)
        loss_ref[0, 0] = (loss_ref[0, 0] + s) / numk


def kernel(pred_similarities, regions_mask, kernels_mask, text_mask_ndi_labels, kernel_mask_ndi_labels):
    B, C, H, W = pred_similarities.shape

    img_spec = lambda: pl.BlockSpec((1, 1, H, W), lambda b: (b, 0, 0, 0))

    loss = pl.pallas_call(
        _body,
        grid=(B,),
        in_specs=[
            pl.BlockSpec((1, C, H, W), lambda b: (b, 0, 0, 0)),
            img_spec(),
            img_spec(),
            img_spec(),
            img_spec(),
        ],
        out_specs=pl.BlockSpec(memory_space=pltpu.SMEM),
        out_shape=jax.ShapeDtypeStruct((1, 1), jnp.float32),
    )(pred_similarities, regions_mask, kernels_mask, text_mask_ndi_labels, kernel_mask_ndi_labels)

    return loss[0, 0]
